# Initial kernel scaffold; baseline (speedup 1.0000x reference)
import jax
import jax.numpy as jnp
from jax.experimental import pallas as pl


def kernel(windows, W1, b1, W2, b2):
    raise NotImplementedError("write your pallas kernel here")


# speedup vs baseline: 25.6444x; 25.6444x over previous
import jax
import jax.numpy as jnp
from jax.experimental import pallas as pl

_LOCAL = 8
_K = 64


def _knn_mlp_kernel(xs_ref, ys_ref, zs_ref, W1_ref, b1_ref, W2_ref, b2_ref,
                    out_ref):
    x = xs_ref[...]
    y = ys_ref[...]
    z = zs_ref[...]
    B = x.shape[0]

    dx = x[:, :, None] - x[:, None, :]
    dy = y[:, :, None] - y[:, None, :]
    dz = z[:, :, None] - z[:, None, :]
    d2 = dx * dx + dy * dy + dz * dz

    iota = jax.lax.broadcasted_iota(jnp.int32, (B, _K, _K), 2)

    W1 = W1_ref[...]
    b1 = b1_ref[0]
    W2 = W2_ref[...]
    b2 = b2_ref[0]

    u = (x[..., None] * W1[0] + y[..., None] * W1[1]
         + z[..., None] * W1[2])

    d2m = d2
    sels = []
    m = None
    for _ in range(_LOCAL):
        m = jnp.min(d2m, axis=-1, keepdims=True)
        cand = jnp.where(d2m == m, iota, jnp.int32(_K))
        fi = jnp.min(cand, axis=-1, keepdims=True)
        sel = iota == fi
        sels.append(sel)
        d2m = jnp.where(sel, jnp.float32(jnp.inf), d2m)

    denom = jnp.sqrt(m) + jnp.float32(1e-8)

    feat = None
    for j in range(_LOCAL):
        sel_f = sels[j].astype(jnp.float32)
        gath = jax.lax.dot_general(
            sel_f, u, (((2,), (1,)), ((0,), (0,))),
            preferred_element_type=jnp.float32)
        h1 = jnp.maximum((gath - u) / denom + b1, 0.0)
        h2 = jax.lax.dot_general(
            h1, W2, (((2,), (0,)), ((), ())),
            preferred_element_type=jnp.float32) + b2
        h2 = jnp.maximum(h2, 0.0)
        feat = h2 if feat is None else jnp.maximum(feat, h2)

    out_ref[...] = feat


def kernel(windows, W1, b1, W2, b2):
    M, K, _ = windows.shape
    xs = windows[:, :, 0]
    ys = windows[:, :, 1]
    zs = windows[:, :, 2]
    b1r = b1.reshape(1, 16)
    b2r = b2.reshape(1, 32)

    B = 8
    grid = (M // B,)
    out = pl.pallas_call(
        _knn_mlp_kernel,
        grid=grid,
        in_specs=[
            pl.BlockSpec((B, K), lambda i: (i, 0)),
            pl.BlockSpec((B, K), lambda i: (i, 0)),
            pl.BlockSpec((B, K), lambda i: (i, 0)),
            pl.BlockSpec((3, 16), lambda i: (0, 0)),
            pl.BlockSpec((1, 16), lambda i: (0, 0)),
            pl.BlockSpec((16, 32), lambda i: (0, 0)),
            pl.BlockSpec((1, 32), lambda i: (0, 0)),
        ],
        out_specs=pl.BlockSpec((B, K, 32), lambda i: (i, 0, 0)),
        out_shape=jax.ShapeDtypeStruct((M, K, 32), jnp.float32),
    )(xs, ys, zs, W1, b1r, W2, b2r)
    return out


# speedup vs baseline: 52.1623x; 2.0341x over previous
import jax
import jax.numpy as jnp
from jax.experimental import pallas as pl

_LOCAL = 8
_K = 64


def _knn_mlp_kernel(xs_ref, ys_ref, zs_ref, W1_ref, b1_ref, W2_ref, b2_ref,
                    out_ref):
    x = xs_ref[...]
    y = ys_ref[...]
    z = zs_ref[...]
    B = x.shape[0]

    dx = x[:, :, None] - x[:, None, :]
    dy = y[:, :, None] - y[:, None, :]
    dz = z[:, :, None] - z[:, None, :]
    d2 = dx * dx + dy * dy + dz * dz

    iota = jax.lax.broadcasted_iota(jnp.int32, (B, _K, _K), 1).astype(jnp.float32)

    W1 = W1_ref[...]
    b1 = b1_ref[0]
    W2 = W2_ref[...]
    b2 = b2_ref[0]

    u = (x[..., None] * W1[0] + y[..., None] * W1[1]
         + z[..., None] * W1[2])

    d2m = d2
    sels = []
    m = None
    for _ in range(_LOCAL):
        m = jnp.min(d2m, axis=1, keepdims=True)
        cand = jnp.where(d2m == m, iota, jnp.float32(_K))
        fi = jnp.min(cand, axis=1, keepdims=True)
        sel = iota == fi
        sels.append(sel)
        d2m = jnp.where(sel, jnp.float32(jnp.inf), d2m)

    denom = jnp.sqrt(m) + jnp.float32(1e-8)
    denom_q = denom[:, 0, :, None]

    feat = None
    for j in range(_LOCAL):
        sel_f = sels[j].astype(jnp.float32)
        gath = jax.lax.dot_general(
            sel_f, u, (((1,), (1,)), ((0,), (0,))),
            preferred_element_type=jnp.float32)
        h1 = jnp.maximum((gath - u) / denom_q + b1, 0.0)
        h2 = jax.lax.dot_general(
            h1, W2, (((2,), (0,)), ((), ())),
            preferred_element_type=jnp.float32) + b2
        h2 = jnp.maximum(h2, 0.0)
        feat = h2 if feat is None else jnp.maximum(feat, h2)

    out_ref[...] = feat


def kernel(windows, W1, b1, W2, b2):
    M, K, _ = windows.shape
    xs = windows[:, :, 0]
    ys = windows[:, :, 1]
    zs = windows[:, :, 2]
    b1r = b1.reshape(1, 16)
    b2r = b2.reshape(1, 32)

    B = 8
    grid = (M // B,)
    out = pl.pallas_call(
        _knn_mlp_kernel,
        grid=grid,
        in_specs=[
            pl.BlockSpec((B, K), lambda i: (i, 0)),
            pl.BlockSpec((B, K), lambda i: (i, 0)),
            pl.BlockSpec((B, K), lambda i: (i, 0)),
            pl.BlockSpec((3, 16), lambda i: (0, 0)),
            pl.BlockSpec((1, 16), lambda i: (0, 0)),
            pl.BlockSpec((16, 32), lambda i: (0, 0)),
            pl.BlockSpec((1, 32), lambda i: (0, 0)),
        ],
        out_specs=pl.BlockSpec((B, K, 32), lambda i: (i, 0, 0)),
        out_shape=jax.ShapeDtypeStruct((M, K, 32), jnp.float32),
    )(xs, ys, zs, W1, b1r, W2, b2r)
    return out
